# Initial kernel scaffold; baseline (speedup 1.0000x reference)
import jax
import jax.numpy as jnp
from jax.experimental import pallas as pl


def kernel(user_emb, item_emb, edge_src, edge_dst, edge_weight, user_ids):
    raise NotImplementedError("write your pallas kernel here")


# speedup vs baseline: 11.9110x; 11.9110x over previous
import jax
import jax.numpy as jnp
from jax import lax
from jax.experimental import pallas as pl
from jax.experimental.pallas import tpu as pltpu
from jax.experimental.pallas import tpu_sc as plsc

USERS = 25000
ITEMS = 25000
N = USERS + ITEMS
E = 800000
DH = 32
NSUB = 16
EPT = E // NSUB
K = 400
NCH = EPT // K
SCB = 80
NSC = K // SCB
CZ = 200
NZ = N // CZ
UPT = 1024 // NSUB
BU = 64

_GD = lax.GatherDimensionNumbers(
    offset_dims=(), collapsed_slice_dims=(0,), start_index_map=(0,))


def _lane_splat(vec, e):
    idx = jnp.full((16, 1), e, jnp.int32)
    return lax.gather(vec, idx, _GD, (1,),
                      mode=lax.GatherScatterMode.PROMISE_IN_BOUNDS)


def _gcn_body(table, srcr, dstr, wr, uids, h1, h2, ssum, up,
              acc, src_i0, src_i1, dst_l0, dst_l1, dst_i0, dst_i1,
              w_v0, w_v1, rows, rows1, uid_v, semG, semI, semS):
    c = lax.axis_index("c")
    s = lax.axis_index("s")
    zv = jnp.zeros((16,), jnp.float32)

    src_b = (src_i0, src_i1)
    dstl_b = (dst_l0, dst_l1)
    dsti_b = (dst_i0, dst_i1)
    w_b = (w_v0, w_v1)
    rows_b = (rows, rows1)

    def run_layer(tbl, out, mul, boff, last):
        boff_v = jnp.full((16,), boff, jnp.int32)

        def zloop(i, carry):
            rows[i, pl.ds(0, 16)] = zv
            rows[i, pl.ds(16, 16)] = zv
            return carry

        lax.fori_loop(0, CZ, zloop, 0)
        for k in range(-(-NZ // NSUB)):
            idx = k * NSUB + s

            @pl.when(idx < NZ)
            def _():
                pltpu.sync_copy(rows.at[pl.ds(0, CZ)],
                                acc.at[pl.ds(idx * CZ, CZ)])

        plsc.subcore_barrier()

        def issue_idx(g, p):
            base = s * EPT + g * K
            pltpu.async_copy(srcr.at[pl.ds(base, K)], src_b[p], semI)
            pltpu.async_copy(dstr.at[pl.ds(base, K)], dstl_b[p], semI)
            pltpu.async_copy(wr.at[pl.ds(base, K)], w_b[p], semI)

        def wait_idx(p):
            for ref, hbm in ((src_b[p], srcr), (dstl_b[p], dstr),
                             (w_b[p], wr)):
                pltpu.make_async_copy(hbm.at[pl.ds(0, K)], ref, semI).wait()

        def adjust(p):
            def adj(j, cy):
                sl = pl.ds(j * 16, 16)
                src_b[p][sl] = src_b[p][sl] * mul + boff_v
                return cy

            lax.fori_loop(0, K // 16, adj, 0)

        def repack(p):
            def rep(b, cy):
                for h in range(SCB // 16):
                    dsti_b[p][b, pl.ds(h * 16, 16)] = (
                        dstl_b[p][pl.ds(b * SCB + h * 16, 16)])
                return cy

            lax.fori_loop(0, NSC, rep, 0)

        def wait_scatters(p):
            for b in range(NSC):
                pltpu.make_async_copy(rows_b[p].at[pl.ds(b * SCB, SCB)],
                                      acc.at[dsti_b[p].at[b]], semS).wait()

        def wmul(p):
            def wm(g, cy):
                wv = w_b[p][pl.ds(g * 16, 16)]
                for e in range(16):
                    ws = _lane_splat(wv, e)
                    r = g * 16 + e
                    rows_b[p][r, pl.ds(0, 16)] = (
                        rows_b[p][r, pl.ds(0, 16)] * ws)
                    rows_b[p][r, pl.ds(16, 16)] = (
                        rows_b[p][r, pl.ds(16, 16)] * ws)
                return cy

            lax.fori_loop(0, K // 16, wm, 0)

        def process(g, p, first_chunk=False, stop_idx=False,
                    last_chunk=False):
            if not last_chunk:
                wait_idx(1 - p)
                adjust(1 - p)
            if not first_chunk:
                wait_scatters(1 - p)
            if not last_chunk:
                repack(1 - p)
                pltpu.async_copy(tbl.at[src_b[1 - p]], rows_b[1 - p], semG)
            pltpu.make_async_copy(tbl.at[pl.ds(0, K)], rows_b[p], semG).wait()
            wmul(p)
            for b in range(NSC):
                pltpu.async_copy(rows_b[p].at[pl.ds(b * SCB, SCB)],
                                 acc.at[dsti_b[p].at[b]], semS, add=True)
            if not (stop_idx or last_chunk):
                issue_idx(g + 2, p)

        base0 = s * EPT
        pltpu.sync_copy(srcr.at[pl.ds(base0, K)], src_i0)
        pltpu.sync_copy(dstr.at[pl.ds(base0, K)], dst_l0)
        pltpu.sync_copy(wr.at[pl.ds(base0, K)], w_v0)
        adjust(0)
        repack(0)
        issue_idx(1, 1)
        pltpu.async_copy(tbl.at[src_i0], rows, semG)
        process(0, 0, first_chunk=True)

        def chunk_pair(i, carry):
            process(2 * i + 1, 1)
            process(2 * i + 2, 0)
            return carry

        lax.fori_loop(0, (NCH - 3) // 2, chunk_pair, 0)
        process(NCH - 2, 1, stop_idx=True)
        process(NCH - 1, 0, last_chunk=True)
        wait_scatters(0)
        plsc.subcore_barrier()

        for k in range(-(-NZ // NSUB)):
            idx = k * NSUB + s

            @pl.when(idx < NZ)
            def _():
                off = c * N + idx * CZ
                if last:
                    pltpu.sync_copy(acc.at[pl.ds(idx * CZ, CZ)],
                                    rows.at[pl.ds(0, CZ)])
                    for prev in (h1, h2):
                        pltpu.sync_copy(prev.at[pl.ds(off, CZ)],
                                        rows.at[pl.ds(CZ, CZ)])

                        def addrow(i, cy):
                            for h in range(2):
                                sl = pl.ds(h * 16, 16)
                                rows[i, sl] = rows[i, sl] + rows[CZ + i, sl]
                            return cy

                        lax.fori_loop(0, CZ, addrow, 0)
                    pltpu.sync_copy(rows.at[pl.ds(0, CZ)],
                                    out.at[pl.ds(off, CZ)])
                else:
                    pltpu.sync_copy(acc.at[pl.ds(idx * CZ, CZ)],
                                    out.at[pl.ds(off, CZ)])

        plsc.subcore_barrier()

    run_layer(table, h1, 2, c, False)
    run_layer(h1, h2, 1, c * N, False)
    run_layer(h2, ssum, 1, c * N, True)

    pltpu.sync_copy(uids.at[pl.ds(s * UPT, UPT)], uid_v)
    off_v = jnp.full((16,), c * N, jnp.int32)

    def uadj(j, cy):
        sl = pl.ds(j * 16, 16)
        uid_v[sl] = uid_v[sl] + off_v
        return cy

    lax.fori_loop(0, UPT // 16, uadj, 0)
    pltpu.async_copy(ssum.at[uid_v], rows.at[pl.ds(0, UPT)], semG).wait()
    pltpu.sync_copy(rows.at[pl.ds(0, UPT)], up.at[c, pl.ds(s * UPT, UPT)])


def _make_gcn():
    mesh = plsc.VectorSubcoreMesh(core_axis_name="c", subcore_axis_name="s")
    emb = jax.ShapeDtypeStruct((2 * N, DH), jnp.float32)
    return pl.kernel(
        _gcn_body,
        out_type=(emb, emb, emb,
                  jax.ShapeDtypeStruct((2, 1024, DH), jnp.float32)),
        mesh=mesh,
        scratch_types=[
            pltpu.VMEM_SHARED((N, DH), jnp.float32),
            pltpu.VMEM((K,), jnp.int32),
            pltpu.VMEM((K,), jnp.int32),
            pltpu.VMEM((K,), jnp.int32),
            pltpu.VMEM((K,), jnp.int32),
            pltpu.VMEM((NSC, SCB), jnp.int32),
            pltpu.VMEM((NSC, SCB), jnp.int32),
            pltpu.VMEM((K,), jnp.float32),
            pltpu.VMEM((K,), jnp.float32),
            pltpu.VMEM((K, DH), jnp.float32),
            pltpu.VMEM((K, DH), jnp.float32),
            pltpu.VMEM((UPT,), jnp.int32),
            pltpu.SemaphoreType.DMA,
            pltpu.SemaphoreType.DMA,
            pltpu.SemaphoreType.DMA,
        ],
        compiler_params=pltpu.CompilerParams(use_tc_tiling_on_sc=False),
        name="lightgcn_gcn3",
    )


def _mm_body(u0, u1, i0, i1, out):
    third = jnp.float32(1.0 / 3.0)
    r = lax.dot_general(u0[...] * third, i0[...] * third,
                        (((1,), (1,)), ((), ())),
                        preferred_element_type=jnp.float32)
    r = r + lax.dot_general(u1[...] * third, i1[...] * third,
                            (((1,), (1,)), ((), ())),
                            preferred_element_type=jnp.float32)
    out[...] = jax.nn.sigmoid(r)


def _rating(u0, u1, ssum):
    ub = pl.BlockSpec((BU, DH), lambda i: (i, 0))
    bs0 = pl.BlockSpec((ITEMS, DH), lambda i: (USERS // ITEMS, 0))
    bs1 = pl.BlockSpec((ITEMS, DH), lambda i: ((N + USERS) // ITEMS, 0))
    f = pl.pallas_call(
        _mm_body,
        grid=(1024 // BU,),
        in_specs=[ub, ub, bs0, bs1],
        out_specs=pl.BlockSpec((BU, ITEMS), lambda i: (i, 0)),
        out_shape=jax.ShapeDtypeStruct((1024, ITEMS), jnp.float32),
    )
    return f(u0, u1, ssum, ssum)


@jax.jit
def kernel(user_emb, item_emb, edge_src, edge_dst, edge_weight, user_ids):
    table0 = jnp.concatenate([user_emb, item_emb], axis=0).reshape(2 * N, DH)
    src = edge_src.astype(jnp.int32)
    dst = edge_dst.astype(jnp.int32)
    w = edge_weight
    uids = user_ids.astype(jnp.int32)

    _h1, _h2, ssum, up = _make_gcn()(table0, src, dst, w, uids)
    return _rating(up[0], up[1], ssum)


# speedup vs baseline: 12.1636x; 1.0212x over previous
import jax
import jax.numpy as jnp
from jax import lax
from jax.experimental import pallas as pl
from jax.experimental.pallas import tpu as pltpu
from jax.experimental.pallas import tpu_sc as plsc

USERS = 25000
ITEMS = 25000
N = USERS + ITEMS
E = 800000
DH = 32
NSUB = 16
EPT = E // NSUB
K = 400
NCH = EPT // K
SCB = 80
NSC = K // SCB
CZ = 200
NZ = N // CZ
UPT = 1024 // NSUB
BU = 128

_GD = lax.GatherDimensionNumbers(
    offset_dims=(), collapsed_slice_dims=(0,), start_index_map=(0,))


def _lane_splat(vec, e):
    idx = jnp.full((16, 1), e, jnp.int32)
    return lax.gather(vec, idx, _GD, (1,),
                      mode=lax.GatherScatterMode.PROMISE_IN_BOUNDS)


def _gcn_body(table, srcr, dstr, wr, uids, h1, h2, ssum, up,
              acc, src_i0, src_i1, dst_l0, dst_l1, dst_i0, dst_i1,
              w_v0, w_v1, rows, rows1, uid_v, semG, semI, semS):
    c = lax.axis_index("c")
    s = lax.axis_index("s")
    zv = jnp.zeros((16,), jnp.float32)

    src_b = (src_i0, src_i1)
    dstl_b = (dst_l0, dst_l1)
    dsti_b = (dst_i0, dst_i1)
    w_b = (w_v0, w_v1)
    rows_b = (rows, rows1)

    def run_layer(tbl, out, mul, boff, last):
        boff_v = jnp.full((16,), boff, jnp.int32)

        def zloop(i, carry):
            rows[i, pl.ds(0, 16)] = zv
            rows[i, pl.ds(16, 16)] = zv
            return carry

        lax.fori_loop(0, CZ, zloop, 0)
        for k in range(-(-NZ // NSUB)):
            idx = k * NSUB + s

            @pl.when(idx < NZ)
            def _():
                pltpu.sync_copy(rows.at[pl.ds(0, CZ)],
                                acc.at[pl.ds(idx * CZ, CZ)])

        plsc.subcore_barrier()

        def issue_idx(g, p):
            base = s * EPT + g * K
            pltpu.async_copy(srcr.at[pl.ds(base, K)], src_b[p], semI)
            pltpu.async_copy(dstr.at[pl.ds(base, K)], dstl_b[p], semI)
            pltpu.async_copy(wr.at[pl.ds(base, K)], w_b[p], semI)

        def wait_idx(p):
            for ref, hbm in ((src_b[p], srcr), (dstl_b[p], dstr),
                             (w_b[p], wr)):
                pltpu.make_async_copy(hbm.at[pl.ds(0, K)], ref, semI).wait()

        def adjust(p):
            def adj(j, cy):
                sl = pl.ds(j * 16, 16)
                src_b[p][sl] = src_b[p][sl] * mul + boff_v
                return cy

            lax.fori_loop(0, K // 16, adj, 0)

        def repack(p):
            def rep(b, cy):
                for h in range(SCB // 16):
                    dsti_b[p][b, pl.ds(h * 16, 16)] = (
                        dstl_b[p][pl.ds(b * SCB + h * 16, 16)])
                return cy

            lax.fori_loop(0, NSC, rep, 0)

        def wait_scatters(p):
            for b in range(NSC):
                pltpu.make_async_copy(rows_b[p].at[pl.ds(b * SCB, SCB)],
                                      acc.at[dsti_b[p].at[b]], semS).wait()

        def wmul(p):
            def wm(g, cy):
                wv = w_b[p][pl.ds(g * 16, 16)]
                for e in range(16):
                    ws = _lane_splat(wv, e)
                    r = g * 16 + e
                    rows_b[p][r, pl.ds(0, 16)] = (
                        rows_b[p][r, pl.ds(0, 16)] * ws)
                    rows_b[p][r, pl.ds(16, 16)] = (
                        rows_b[p][r, pl.ds(16, 16)] * ws)
                return cy

            lax.fori_loop(0, K // 16, wm, 0)

        def process(g, p, first_chunk=False, stop_idx=False,
                    last_chunk=False):
            if not last_chunk:
                wait_idx(1 - p)
                adjust(1 - p)
            if not first_chunk:
                wait_scatters(1 - p)
            if not last_chunk:
                repack(1 - p)
                pltpu.async_copy(tbl.at[src_b[1 - p]], rows_b[1 - p], semG)
            pltpu.make_async_copy(tbl.at[pl.ds(0, K)], rows_b[p], semG).wait()
            wmul(p)
            for b in range(NSC):
                pltpu.async_copy(rows_b[p].at[pl.ds(b * SCB, SCB)],
                                 acc.at[dsti_b[p].at[b]], semS, add=True)
            if not (stop_idx or last_chunk):
                issue_idx(g + 2, p)

        base0 = s * EPT
        pltpu.sync_copy(srcr.at[pl.ds(base0, K)], src_i0)
        pltpu.sync_copy(dstr.at[pl.ds(base0, K)], dst_l0)
        pltpu.sync_copy(wr.at[pl.ds(base0, K)], w_v0)
        adjust(0)
        repack(0)
        issue_idx(1, 1)
        pltpu.async_copy(tbl.at[src_i0], rows, semG)
        process(0, 0, first_chunk=True)

        def chunk_pair(i, carry):
            process(2 * i + 1, 1)
            process(2 * i + 2, 0)
            return carry

        lax.fori_loop(0, (NCH - 3) // 2, chunk_pair, 0)
        process(NCH - 2, 1, stop_idx=True)
        process(NCH - 1, 0, last_chunk=True)
        wait_scatters(0)
        plsc.subcore_barrier()

        for k in range(-(-NZ // NSUB)):
            idx = k * NSUB + s

            @pl.when(idx < NZ)
            def _():
                off = c * N + idx * CZ
                if last:
                    pltpu.sync_copy(acc.at[pl.ds(idx * CZ, CZ)],
                                    rows.at[pl.ds(0, CZ)])
                    for prev in (h1, h2):
                        pltpu.sync_copy(prev.at[pl.ds(off, CZ)],
                                        rows.at[pl.ds(CZ, CZ)])

                        def addrow(i, cy):
                            for h in range(2):
                                sl = pl.ds(h * 16, 16)
                                rows[i, sl] = rows[i, sl] + rows[CZ + i, sl]
                            return cy

                        lax.fori_loop(0, CZ, addrow, 0)
                    pltpu.sync_copy(rows.at[pl.ds(0, CZ)],
                                    out.at[pl.ds(off, CZ)])
                else:
                    pltpu.sync_copy(acc.at[pl.ds(idx * CZ, CZ)],
                                    out.at[pl.ds(off, CZ)])

        plsc.subcore_barrier()

    run_layer(table, h1, 2, c, False)
    run_layer(h1, h2, 1, c * N, False)
    run_layer(h2, ssum, 1, c * N, True)

    pltpu.sync_copy(uids.at[pl.ds(s * UPT, UPT)], uid_v)
    off_v = jnp.full((16,), c * N, jnp.int32)

    def uadj(j, cy):
        sl = pl.ds(j * 16, 16)
        uid_v[sl] = uid_v[sl] + off_v
        return cy

    lax.fori_loop(0, UPT // 16, uadj, 0)
    pltpu.async_copy(ssum.at[uid_v], rows.at[pl.ds(0, UPT)], semG).wait()
    pltpu.sync_copy(rows.at[pl.ds(0, UPT)], up.at[c, pl.ds(s * UPT, UPT)])


def _make_gcn():
    mesh = plsc.VectorSubcoreMesh(core_axis_name="c", subcore_axis_name="s")
    emb = jax.ShapeDtypeStruct((2 * N, DH), jnp.float32)
    return pl.kernel(
        _gcn_body,
        out_type=(emb, emb, emb,
                  jax.ShapeDtypeStruct((2, 1024, DH), jnp.float32)),
        mesh=mesh,
        scratch_types=[
            pltpu.VMEM_SHARED((N, DH), jnp.float32),
            pltpu.VMEM((K,), jnp.int32),
            pltpu.VMEM((K,), jnp.int32),
            pltpu.VMEM((K,), jnp.int32),
            pltpu.VMEM((K,), jnp.int32),
            pltpu.VMEM((NSC, SCB), jnp.int32),
            pltpu.VMEM((NSC, SCB), jnp.int32),
            pltpu.VMEM((K,), jnp.float32),
            pltpu.VMEM((K,), jnp.float32),
            pltpu.VMEM((K, DH), jnp.float32),
            pltpu.VMEM((K, DH), jnp.float32),
            pltpu.VMEM((UPT,), jnp.int32),
            pltpu.SemaphoreType.DMA,
            pltpu.SemaphoreType.DMA,
            pltpu.SemaphoreType.DMA,
        ],
        compiler_params=pltpu.CompilerParams(use_tc_tiling_on_sc=False),
        name="lightgcn_gcn3",
    )


def _mm_body(u0, u1, i0, i1, out):
    third = jnp.float32(1.0 / 3.0)
    r = lax.dot_general(u0[...] * third, i0[...] * third,
                        (((1,), (1,)), ((), ())),
                        preferred_element_type=jnp.float32)
    r = r + lax.dot_general(u1[...] * third, i1[...] * third,
                            (((1,), (1,)), ((), ())),
                            preferred_element_type=jnp.float32)
    out[...] = jax.nn.sigmoid(r)


def _rating(u0, u1, ssum):
    ub = pl.BlockSpec((BU, DH), lambda i: (i, 0))
    bs0 = pl.BlockSpec((ITEMS, DH), lambda i: (USERS // ITEMS, 0))
    bs1 = pl.BlockSpec((ITEMS, DH), lambda i: ((N + USERS) // ITEMS, 0))
    f = pl.pallas_call(
        _mm_body,
        grid=(1024 // BU,),
        in_specs=[ub, ub, bs0, bs1],
        out_specs=pl.BlockSpec((BU, ITEMS), lambda i: (i, 0)),
        out_shape=jax.ShapeDtypeStruct((1024, ITEMS), jnp.float32),
        compiler_params=pltpu.CompilerParams(
            vmem_limit_bytes=64 * 1024 * 1024),
    )
    return f(u0, u1, ssum, ssum)


@jax.jit
def kernel(user_emb, item_emb, edge_src, edge_dst, edge_weight, user_ids):
    table0 = jnp.concatenate([user_emb, item_emb], axis=0).reshape(2 * N, DH)
    src = edge_src.astype(jnp.int32)
    dst = edge_dst.astype(jnp.int32)
    w = edge_weight
    uids = user_ids.astype(jnp.int32)

    _h1, _h2, ssum, up = _make_gcn()(table0, src, dst, w, uids)
    return _rating(up[0], up[1], ssum)
